# Initial kernel scaffold; baseline (speedup 1.0000x reference)
import jax
import jax.numpy as jnp
from jax.experimental import pallas as pl


def kernel(decoder_output, context_vector, decoder_input_embed, vocab_logits, cross_attn_weights, src_input_ids, W_gen, b_gen):
    raise NotImplementedError("write your pallas kernel here")


# speedup vs baseline: 2.3977x; 2.3977x over previous
import functools

import jax
import jax.numpy as jnp
from jax import lax
from jax.experimental import pallas as pl
from jax.experimental.pallas import tpu as pltpu
from jax.experimental.pallas import tpu_sc as plsc

B, T, S, D, V = 8, 128, 1024, 1024, 50000
BT = B * T
R2 = 16
L = 16


def _pgen_cc_body(ids_ref, dec_ref, ctx_ref, emb_ref, cross_ref, w_ref, b_ref,
                  pg_ref, cc_ref):
    z = (
        jnp.dot(dec_ref[...], w_ref[0:D, :], preferred_element_type=jnp.float32)
        + jnp.dot(ctx_ref[...], w_ref[D:2 * D, :], preferred_element_type=jnp.float32)
        + jnp.dot(emb_ref[...], w_ref[2 * D:3 * D, :], preferred_element_type=jnp.float32)
        + b_ref[...]
    )
    pg = jax.nn.sigmoid(z)
    pg_ref[...] = pg
    cd = (1.0 - pg) * cross_ref[...]
    ids_f = ids_ref[0].astype(jnp.float32)
    eq = (jnp.transpose(ids_f) == ids_f).astype(jnp.float32)
    cc_ref[...] = jnp.dot(cd, eq, preferred_element_type=jnp.float32)


def _softmax_scale_body(pg_ref, logits_ref, out_ref):
    x = logits_ref[...]
    m = jnp.max(x, axis=-1, keepdims=True)
    e = jnp.exp(x - m)
    s = jnp.sum(e, axis=-1, keepdims=True)
    out_ref[...] = e * (pg_ref[...] / s)


def _sc_scatter_body(rows_per_w, nc, final_ref, cc_hbm, ids_hbm,
                     ids_v, cc_v, addr_v, vals_v, sem):
    wid = lax.axis_index("s") * nc + lax.axis_index("c")
    pltpu.sync_copy(ids_hbm.at[wid // (T // rows_per_w)], ids_v)
    r0 = wid * rows_per_w

    @pl.loop(0, rows_per_w)
    def _row(i):
        r = r0 + i
        pltpu.sync_copy(cc_hbm.at[r], cc_v)
        base = r * V
        for j in range(8):
            for k in range(8):
                c = j * 8 + k
                addr_v[j, pl.ds(k * L, L)] = ids_v[pl.ds(c * L, L)] + base
        gathers = [
            pltpu.async_copy(final_ref.at[addr_v.at[j]], vals_v.at[j], sem)
            for j in range(8)
        ]
        for g in gathers:
            g.wait()
        for j in range(8):
            for k in range(8):
                c = j * 8 + k
                vals_v[j, pl.ds(k * L, L)] = (
                    vals_v[j, pl.ds(k * L, L)] + cc_v[pl.ds(c * L, L)]
                )
        scatters = [
            pltpu.async_copy(vals_v.at[j], final_ref.at[addr_v.at[j]], sem)
            for j in range(8)
        ]
        for sc in scatters:
            sc.wait()


def kernel(decoder_output, context_vector, decoder_input_embed, vocab_logits,
           cross_attn_weights, src_input_ids, W_gen, b_gen):
    dec2 = decoder_output.reshape(BT, D)
    ctx2 = context_vector.reshape(BT, D)
    emb2 = decoder_input_embed.reshape(BT, D)
    logits2 = vocab_logits.reshape(BT, V)
    cross2 = cross_attn_weights.reshape(BT, S)
    ids32 = src_input_ids.astype(jnp.int32)
    ids3d = ids32.reshape(B, 1, S)
    b2 = b_gen.reshape(1, 1)

    pg, cc = pl.pallas_call(
        _pgen_cc_body,
        grid=(B,),
        in_specs=[
            pl.BlockSpec((1, 1, S), lambda b: (b, 0, 0)),
            pl.BlockSpec((T, D), lambda b: (b, 0)),
            pl.BlockSpec((T, D), lambda b: (b, 0)),
            pl.BlockSpec((T, D), lambda b: (b, 0)),
            pl.BlockSpec((T, S), lambda b: (b, 0)),
            pl.BlockSpec((3 * D, 1), lambda b: (0, 0)),
            pl.BlockSpec((1, 1), lambda b: (0, 0)),
        ],
        out_specs=[
            pl.BlockSpec((T, 1), lambda b: (b, 0)),
            pl.BlockSpec((T, S), lambda b: (b, 0)),
        ],
        out_shape=[
            jax.ShapeDtypeStruct((BT, 1), jnp.float32),
            jax.ShapeDtypeStruct((BT, S), jnp.float32),
        ],
    )(ids3d, dec2, ctx2, emb2, cross2, W_gen, b2)

    final_base = pl.pallas_call(
        _softmax_scale_body,
        grid=(BT // R2,),
        in_specs=[
            pl.BlockSpec((R2, 1), lambda i: (i, 0)),
            pl.BlockSpec((R2, V), lambda i: (i, 0)),
        ],
        out_specs=pl.BlockSpec((R2, V), lambda i: (i, 0)),
        out_shape=jax.ShapeDtypeStruct((BT, V), jnp.float32),
    )(pg, logits2)

    info = plsc.get_sparse_core_info()
    nw = info.num_cores * info.num_subcores
    rows_per_w = BT // nw
    mesh = plsc.VectorSubcoreMesh(core_axis_name="c", subcore_axis_name="s")
    sc_call = pl.kernel(
        functools.partial(_sc_scatter_body, rows_per_w, info.num_cores),
        out_type=(),
        mesh=mesh,
        scratch_types=[
            pltpu.VMEM((S,), jnp.int32),
            pltpu.VMEM((S,), jnp.float32),
            pltpu.VMEM((8, 128), jnp.int32),
            pltpu.VMEM((8, 128), jnp.float32),
            pltpu.SemaphoreType.DMA,
        ],
    )
    final_ref = jax.new_ref(final_base.reshape(BT * V))
    sc_call(final_ref, cc, ids32)
    final = jax.freeze(final_ref).reshape(B, T, V)
    return (final, pg.reshape(B, T, 1))
